# baseline (speedup 1.0000x reference)
import functools

import jax
import jax.numpy as jnp
from jax import lax
from jax.experimental import pallas as pl
from jax.experimental.pallas import tpu as pltpu
from jax.experimental.pallas import tpu_sc as plsc

MAX_LEN = 200
DIM = 64
BATCH = 16384
NUM_CORES = 2
NUM_SUBCORES = 16
NUM_WORKERS = NUM_CORES * NUM_SUBCORES
LANES = 16

SB = 5
DQ = 16
BW = 512
UNITS = (MAX_LEN // SB) * (DIM // DQ)
UNITS_PER_W = UNITS // NUM_WORKERS
NCHUNKS = BATCH // BW
FIRE = 16


@jax.jit
def _sc_pos_broadcast(w):
    mesh = plsc.VectorSubcoreMesh(
        core_axis_name="c", subcore_axis_name="s",
        num_cores=NUM_CORES, num_subcores=NUM_SUBCORES)

    @functools.partial(
        pl.kernel,
        mesh=mesh,
        out_type=jax.ShapeDtypeStruct((MAX_LEN, DIM, BATCH), jnp.float32),
        scratch_types=[
            pltpu.VMEM((MAX_LEN, DIM), jnp.float32),
            pltpu.VMEM((SB, DQ, BW), jnp.float32),
            pltpu.VMEM((SB, DQ, BW), jnp.float32),
            pltpu.SemaphoreType.DMA,
            pltpu.SemaphoreType.DMA,
        ],
        compiler_params=pltpu.CompilerParams(use_tc_tiling_on_sc=True),
    )
    def k(w_hbm, out_hbm, w_v, src_a, src_b, sem_a, sem_b):
        wid = lax.axis_index("s") * NUM_CORES + lax.axis_index("c")
        pltpu.sync_copy(w_hbm, w_v)
        bufs = (src_a, src_b)
        sems = (sem_a, sem_b)

        def unit_coords(i):
            u = wid * UNITS_PER_W + i
            s0 = (u // (DIM // DQ)) * SB
            d0 = (u % (DIM // DQ)) * DQ
            return s0, d0

        def build(i, src):
            s0, d0 = unit_coords(i)

            def build_row(si, c2):
                vec = jnp.maximum(w_v[s0 + si, pl.ds(d0, DQ)], 0.0)
                for di in range(DQ):
                    splat = jnp.full((LANES,), vec[di], dtype=jnp.float32)
                    for c in range(BW // LANES):
                        src[si, di, pl.ds(c * LANES, LANES)] = splat
                return c2

            lax.fori_loop(0, SB, build_row, 0)

        def fire(i, src, sem):
            s0, d0 = unit_coords(i)

            def ring(j, c2):
                pltpu.async_copy(
                    src,
                    out_hbm.at[pl.ds(s0, SB), pl.ds(d0, DQ),
                               pl.ds(j * BW, BW)],
                    sem)

                @pl.when(j >= FIRE)
                def _():
                    pltpu.make_async_copy(
                        src,
                        out_hbm.at[pl.ds(s0, SB), pl.ds(d0, DQ),
                                   pl.ds(0, BW)],
                        sem).wait()

                return c2

            lax.fori_loop(0, NCHUNKS, ring, 0)

        def drain(i, sem):
            s0, d0 = unit_coords(i)

            def one(j, c2):
                pltpu.make_async_copy(
                    bufs[0],
                    out_hbm.at[pl.ds(s0, SB), pl.ds(d0, DQ), pl.ds(0, BW)],
                    sem).wait()
                return c2

            lax.fori_loop(0, min(FIRE, NCHUNKS), one, 0)

        build(0, bufs[0])
        fire(0, bufs[0], sems[0])
        build(1, bufs[1])
        fire(1, bufs[1], sems[1])
        for i in range(2, UNITS_PER_W):
            p = i % 2
            drain(i - 2, sems[p])
            build(i, bufs[p])
            fire(i, bufs[p], sems[p])
        drain(UNITS_PER_W - 2, sems[(UNITS_PER_W - 2) % 2])
        drain(UNITS_PER_W - 1, sems[(UNITS_PER_W - 1) % 2])

    return k(w)


def kernel(x, embed_weight):
    seq = x.shape[1]
    out = _sc_pos_broadcast(embed_weight[:seq])
    return jnp.transpose(out, (2, 0, 1))


# speedup vs baseline: 1.0153x; 1.0153x over previous
import functools

import jax
import jax.numpy as jnp
from jax import lax
from jax.experimental import pallas as pl
from jax.experimental.pallas import tpu as pltpu
from jax.experimental.pallas import tpu_sc as plsc

MAX_LEN = 200
DIM = 64
BATCH = 16384
NUM_CORES = 2
NUM_SUBCORES = 16
NUM_WORKERS = NUM_CORES * NUM_SUBCORES
LANES = 16

SB = 5
DQ = 16
BW = 512
UNITS = (MAX_LEN // SB) * (DIM // DQ)
UNITS_PER_W = UNITS // NUM_WORKERS
NCHUNKS = BATCH // BW
FIRE = 16


@jax.jit
def _sc_pos_broadcast(w):
    mesh = plsc.VectorSubcoreMesh(
        core_axis_name="c", subcore_axis_name="s",
        num_cores=NUM_CORES, num_subcores=NUM_SUBCORES)

    @functools.partial(
        pl.kernel,
        mesh=mesh,
        out_type=jax.ShapeDtypeStruct((MAX_LEN, DIM, BATCH), jnp.float32),
        scratch_types=[
            pltpu.VMEM((MAX_LEN, DIM), jnp.float32),
            pltpu.VMEM((SB, DQ, BW), jnp.float32),
            pltpu.SemaphoreType.DMA,
        ],
        compiler_params=pltpu.CompilerParams(use_tc_tiling_on_sc=True),
    )
    def k(w_hbm, out_hbm, w_v, src, sem):
        wid = lax.axis_index("s") * NUM_CORES + lax.axis_index("c")
        pltpu.sync_copy(w_hbm, w_v)

        def do_unit(i, carry):
            u = wid * UNITS_PER_W + i
            s0 = (u // (DIM // DQ)) * SB
            d0 = (u % (DIM // DQ)) * DQ

            def build_row(si, c2):
                vec = jnp.maximum(w_v[s0 + si, pl.ds(d0, DQ)], 0.0)
                for di in range(DQ):
                    splat = jnp.full((LANES,), vec[di], dtype=jnp.float32)
                    for c in range(BW // LANES):
                        src[si, di, pl.ds(c * LANES, LANES)] = splat
                return c2

            lax.fori_loop(0, SB, build_row, 0)

            def ring(j, c2):
                pltpu.async_copy(
                    src,
                    out_hbm.at[pl.ds(s0, SB), pl.ds(d0, DQ),
                               pl.ds(j * BW, BW)],
                    sem)

                @pl.when(j >= FIRE)
                def _():
                    pltpu.make_async_copy(
                        src,
                        out_hbm.at[pl.ds(s0, SB), pl.ds(d0, DQ),
                                   pl.ds(0, BW)],
                        sem).wait()

                return c2

            lax.fori_loop(0, NCHUNKS, ring, 0)

            def drain(j, c2):
                pltpu.make_async_copy(
                    src,
                    out_hbm.at[pl.ds(s0, SB), pl.ds(d0, DQ), pl.ds(0, BW)],
                    sem).wait()
                return c2

            lax.fori_loop(0, FIRE, drain, 0)
            return carry

        lax.fori_loop(0, UNITS_PER_W, do_unit, 0)

    return k(w)


def kernel(x, embed_weight):
    seq = x.shape[1]
    out = _sc_pos_broadcast(embed_weight[:seq])
    return jnp.transpose(out, (2, 0, 1))


# speedup vs baseline: 1.0261x; 1.0106x over previous
import functools

import jax
import jax.numpy as jnp
from jax import lax
from jax.experimental import pallas as pl
from jax.experimental.pallas import tpu as pltpu
from jax.experimental.pallas import tpu_sc as plsc

MAX_LEN = 200
DIM = 64
BATCH = 16384
NUM_CORES = 2
NUM_SUBCORES = 16
NUM_WORKERS = NUM_CORES * NUM_SUBCORES
LANES = 16

SB = 5
DQ = 16
BW = 512
UNITS = (MAX_LEN // SB) * (DIM // DQ)
UNITS_PER_W = UNITS // NUM_WORKERS
NCHUNKS = BATCH // BW
FIRE = 16


@jax.jit
def _sc_pos_broadcast(w):
    mesh = plsc.VectorSubcoreMesh(
        core_axis_name="c", subcore_axis_name="s",
        num_cores=NUM_CORES, num_subcores=NUM_SUBCORES)

    @functools.partial(
        pl.kernel,
        mesh=mesh,
        out_type=jax.ShapeDtypeStruct((MAX_LEN, DIM, BATCH), jnp.float32),
        scratch_types=[
            pltpu.VMEM((MAX_LEN, DIM), jnp.float32),
            pltpu.VMEM((SB, DQ, BW), jnp.float32),
            pltpu.SemaphoreType.DMA,
        ],
        compiler_params=pltpu.CompilerParams(use_tc_tiling_on_sc=True),
    )
    def k(w_hbm, out_hbm, w_v, src, sem):
        wid = lax.axis_index("s") * NUM_CORES + lax.axis_index("c")
        pltpu.sync_copy(w_hbm, w_v)

        def do_unit(i, carry):
            u = wid * UNITS_PER_W + i
            s0 = (u // (DIM // DQ)) * SB
            d0 = (u % (DIM // DQ)) * DQ

            def build_row(si, c2):
                vec = jnp.maximum(w_v[s0 + si, pl.ds(d0, DQ)], 0.0)
                for di in range(DQ):
                    splat = jnp.full((LANES,), vec[di], dtype=jnp.float32)
                    for c in range(BW // LANES):
                        src[si, di, pl.ds(c * LANES, LANES)] = splat
                return c2

            lax.fori_loop(0, SB, build_row, 0)

            def ring(j, c2):
                jj = lax.rem(j + wid, NCHUNKS)
                pltpu.async_copy(
                    src,
                    out_hbm.at[pl.ds(s0, SB), pl.ds(d0, DQ),
                               pl.ds(jj * BW, BW)],
                    sem)

                @pl.when(j >= FIRE)
                def _():
                    pltpu.make_async_copy(
                        src,
                        out_hbm.at[pl.ds(s0, SB), pl.ds(d0, DQ),
                                   pl.ds(0, BW)],
                        sem).wait()

                return c2

            lax.fori_loop(0, NCHUNKS, ring, 0)

            def drain(j, c2):
                pltpu.make_async_copy(
                    src,
                    out_hbm.at[pl.ds(s0, SB), pl.ds(d0, DQ), pl.ds(0, BW)],
                    sem).wait()
                return c2

            lax.fori_loop(0, FIRE, drain, 0)
            return carry

        lax.fori_loop(0, UNITS_PER_W, do_unit, 0)

    return k(w)


def kernel(x, embed_weight):
    seq = x.shape[1]
    out = _sc_pos_broadcast(embed_weight[:seq])
    return jnp.transpose(out, (2, 0, 1))
